# Initial kernel scaffold; baseline (speedup 1.0000x reference)
import jax
import jax.numpy as jnp
from jax.experimental import pallas as pl


def kernel(x, edge_index, W, b, a, u):
    raise NotImplementedError("write your pallas kernel here")


# speedup vs baseline: 10.9418x; 10.9418x over previous
import functools

import jax
import jax.numpy as jnp
from jax import lax
from jax.experimental import pallas as pl
from jax.experimental.pallas import tpu as pltpu
from jax.experimental.pallas import tpu_sc as plsc

N = 10000
E = 160000
NF = 256
NH = 512

NP = 10240
EP = 163840
ER = EP // 128
RPT = ER // 16
NSL = NP // 16

_F32 = jnp.float32
_I32 = jnp.int32


def _fast_rsqrt(d):
    ih = plsc.bitcast(d, _I32)
    ih = jnp.int32(0x5F3759DF) - lax.shift_right_logical(ih, 1)
    y = plsc.bitcast(ih, _F32)
    for _ in range(3):
        y = y * (1.5 - 0.5 * d * y * y)
    return y


def _k1_body(x_hbm, dst_hbm, dinv_hbm, xp0_hbm, xp1_hbm,
             idx_v, deg_v, dv_v, tmp_v, xb_v, shd):
    c = lax.axis_index("c")
    s = lax.axis_index("s")
    base = s * NSL

    pltpu.sync_copy(dst_hbm.at[pl.ds(s * RPT, RPT)], idx_v)

    zeros16 = jnp.zeros((16,), _F32)

    def _zero(i, _):
        deg_v[pl.ds(pl.multiple_of(i * 16, 16), 16)] = zeros16
        return 0

    lax.fori_loop(0, NP // 16, _zero, 0)

    ones16 = jnp.ones((16,), _F32)

    def _scat(j, _):
        for k in range(8):
            iv = idx_v[j, pl.ds(k * 16, 16)]
            plsc.addupdate_scatter(deg_v, [iv], ones16)
        return 0

    lax.fori_loop(0, RPT, _scat, 0)

    pltpu.sync_copy(deg_v, shd.at[s])
    plsc.subcore_barrier()

    pltpu.sync_copy(shd.at[0, pl.ds(base, NSL)], dv_v)
    for t in range(1, 16):
        pltpu.sync_copy(shd.at[t, pl.ds(base, NSL)], tmp_v)

        def _acc(i, _):
            sl = pl.ds(pl.multiple_of(i * 16, 16), 16)
            dv_v[sl] = dv_v[sl] + tmp_v[sl]
            return 0

        lax.fori_loop(0, NSL // 16, _acc, 0)

    def _rs(i, _):
        sl = pl.ds(pl.multiple_of(i * 16, 16), 16)
        dv_v[sl] = _fast_rsqrt(dv_v[sl] + 1.0)
        return 0

    lax.fori_loop(0, NSL // 16, _rs, 0)

    @pl.when(c == 0)
    def _():
        pltpu.sync_copy(dv_v, dinv_hbm.at[pl.ds(base, NSL)])

    half = NSL // 2
    for h in range(2):
        r0 = base + h * half
        pltpu.sync_copy(
            x_hbm.at[pl.ds(r0, half), pl.ds(pl.multiple_of(c * 128, 128), 128)],
            xb_v)

        def _scale(i, _):
            ridx = jnp.full((16,), h * half + i, _I32)
            dsp = plsc.load_gather(dv_v, [ridx])
            for k in range(8):
                sl = pl.ds(k * 16, 16)
                xb_v[i, sl] = xb_v[i, sl] * dsp
            return 0

        lax.fori_loop(0, half, _scale, 0)

        @pl.when(c == 0)
        def _():
            pltpu.sync_copy(xb_v, xp0_hbm.at[pl.ds(r0, half)])

        @pl.when(c == 1)
        def _():
            pltpu.sync_copy(xb_v, xp1_hbm.at[pl.ds(r0, half)])


def _k2_body(xp0_hbm, xp1_hbm, src_hbm, dst_hbm, agg0_hbm, agg1_hbm,
             sidx_v, didx_v, buf_v, acc_sh, sem):
    c = lax.axis_index("c")
    s = lax.axis_index("s")

    pltpu.sync_copy(src_hbm.at[pl.ds(s * RPT, RPT)], sidx_v)
    pltpu.sync_copy(dst_hbm.at[pl.ds(s * RPT, RPT)], didx_v)

    zeros16 = jnp.zeros((16,), _F32)

    def _zero(i, _):
        for k in range(8):
            buf_v[i, pl.ds(k * 16, 16)] = zeros16
        return 0

    lax.fori_loop(0, 128, _zero, 0)

    for h in range(NSL // 128):
        pltpu.sync_copy(buf_v, acc_sh.at[pl.ds(s * NSL + h * 128, 128)])
    plsc.subcore_barrier()

    def _edge(j, _):
        @pl.when(c == 0)
        def _():
            pltpu.async_copy(xp0_hbm.at[sidx_v.at[j]], buf_v, sem).wait()

        @pl.when(c == 1)
        def _():
            pltpu.async_copy(xp1_hbm.at[sidx_v.at[j]], buf_v, sem).wait()

        pltpu.sync_copy(buf_v, acc_sh.at[didx_v.at[j]], add=True)
        return 0

    lax.fori_loop(0, RPT, _edge, 0)
    plsc.subcore_barrier()

    for h in range(NSL // 128):
        rows = pl.ds(s * NSL + h * 128, 128)
        pltpu.sync_copy(acc_sh.at[rows], buf_v)

        @pl.when(c == 0)
        def _():
            pltpu.sync_copy(buf_v, agg0_hbm.at[rows])

        @pl.when(c == 1)
        def _():
            pltpu.sync_copy(buf_v, agg1_hbm.at[rows])


def _k3_body(agg0, agg1, xp0, xp1, dinv, w, b2, a2, u2, out_ref):
    w_ = w[...]
    u_ = u2[...]
    wv = jnp.dot(u_, w_, preferred_element_type=_F32)
    nv = jnp.sqrt(jnp.sum(wv * wv))
    v = wv / (nv + 1e-12)
    wv2 = lax.dot_general(v, w_, (((1,), (1,)), ((), ())),
                          preferred_element_type=_F32)
    nu = jnp.sqrt(jnp.sum(wv2 * wv2))
    sigma = jnp.sum(wv2 * wv2) / (nu + 1e-12)

    d = dinv[...]
    t0 = (agg0[...] + xp0[...]) * d
    t1 = (agg1[...] + xp1[...]) * d
    o = (jnp.dot(t0, w_[0:128, :], preferred_element_type=_F32)
         + jnp.dot(t1, w_[128:256, :], preferred_element_type=_F32))
    o = o * (1.0 / sigma) + b2[...]
    al = a2[0, 0]
    out_ref[...] = jnp.where(o >= 0, o, al * o)


@jax.jit
def kernel(x, edge_index, W, b, a, u):
    src = edge_index[0]
    dst = edge_index[1]
    pad = jnp.full((EP - E,), N, _I32)
    srcr = jnp.concatenate([src, pad]).reshape(ER, 128)
    dstr = jnp.concatenate([dst, pad]).reshape(ER, 128)
    x_pad = jnp.pad(x, ((0, NP - N), (0, 0)))

    mesh = plsc.VectorSubcoreMesh(core_axis_name="c", subcore_axis_name="s")

    k1 = pl.kernel(
        _k1_body,
        out_type=(
            jax.ShapeDtypeStruct((NP,), _F32),
            jax.ShapeDtypeStruct((NP, 128), _F32),
            jax.ShapeDtypeStruct((NP, 128), _F32),
        ),
        mesh=mesh,
        scratch_types=[
            pltpu.VMEM((RPT, 128), _I32),
            pltpu.VMEM((NP,), _F32),
            pltpu.VMEM((NSL,), _F32),
            pltpu.VMEM((NSL,), _F32),
            pltpu.VMEM((NSL // 2, 128), _F32),
            pltpu.VMEM_SHARED((16, NP), _F32),
        ],
        compiler_params=pltpu.CompilerParams(needs_layout_passes=False),
    )
    dinv, xp0, xp1 = k1(x_pad, dstr)

    k2 = pl.kernel(
        _k2_body,
        out_type=(
            jax.ShapeDtypeStruct((NP, 128), _F32),
            jax.ShapeDtypeStruct((NP, 128), _F32),
        ),
        mesh=mesh,
        scratch_types=[
            pltpu.VMEM((RPT, 128), _I32),
            pltpu.VMEM((RPT, 128), _I32),
            pltpu.VMEM((128, 128), _F32),
            pltpu.VMEM_SHARED((NP, 128), _F32),
            pltpu.SemaphoreType.DMA,
        ],
        compiler_params=pltpu.CompilerParams(needs_layout_passes=False),
    )
    agg0, agg1 = k2(xp0, xp1, srcr, dstr)

    blk = 512
    grid = NP // blk
    outp = pl.pallas_call(
        _k3_body,
        grid=(grid,),
        in_specs=[
            pl.BlockSpec((blk, 128), lambda i: (i, 0)),
            pl.BlockSpec((blk, 128), lambda i: (i, 0)),
            pl.BlockSpec((blk, 128), lambda i: (i, 0)),
            pl.BlockSpec((blk, 128), lambda i: (i, 0)),
            pl.BlockSpec((blk, 1), lambda i: (i, 0)),
            pl.BlockSpec((NF, NH), lambda i: (0, 0)),
            pl.BlockSpec((1, NH), lambda i: (0, 0)),
            pl.BlockSpec((1, 1), lambda i: (0, 0)),
            pl.BlockSpec((1, NF), lambda i: (0, 0)),
        ],
        out_specs=pl.BlockSpec((blk, NH), lambda i: (i, 0)),
        out_shape=jax.ShapeDtypeStruct((NP, NH), _F32),
    )(agg0, agg1, xp0, xp1, dinv.reshape(NP, 1), W,
      b.reshape(1, NH), a.reshape(1, 1), u.reshape(1, NF))

    return outp[:N]


# speedup vs baseline: 12.3651x; 1.1301x over previous
import functools

import jax
import jax.numpy as jnp
from jax import lax
from jax.experimental import pallas as pl
from jax.experimental.pallas import tpu as pltpu
from jax.experimental.pallas import tpu_sc as plsc

N = 10000
E = 160000
NF = 256
NH = 512

NP = 10240
EP = 163840
ER = EP // 128
RPT = ER // 16
NSL = NP // 16
CHUNK = 16

_F32 = jnp.float32
_I32 = jnp.int32


def _fast_rsqrt(d):
    ih = plsc.bitcast(d, _I32)
    ih = jnp.int32(0x5F3759DF) - lax.shift_right_logical(ih, 1)
    y = plsc.bitcast(ih, _F32)
    for _ in range(3):
        y = y * (1.5 - 0.5 * d * y * y)
    return y


def _k1_body(x_hbm, dst_hbm, dinv_hbm, xp0_hbm, xp1_hbm,
             idx_v, deg_v, dv_v, tmp_v, xb_v, shd):
    c = lax.axis_index("c")
    s = lax.axis_index("s")
    base = s * NSL

    pltpu.sync_copy(dst_hbm.at[pl.ds(s * RPT, RPT)], idx_v)

    zeros16 = jnp.zeros((16,), _F32)

    def _zero(i, _):
        deg_v[pl.ds(pl.multiple_of(i * 16, 16), 16)] = zeros16
        return 0

    lax.fori_loop(0, NP // 16, _zero, 0)

    ones16 = jnp.ones((16,), _F32)

    def _scat(j, _):
        for k in range(8):
            iv = idx_v[j, pl.ds(k * 16, 16)]
            plsc.addupdate_scatter(deg_v, [iv], ones16)
        return 0

    lax.fori_loop(0, RPT, _scat, 0)

    pltpu.sync_copy(deg_v, shd.at[s])
    plsc.subcore_barrier()

    pltpu.sync_copy(shd.at[0, pl.ds(base, NSL)], dv_v)
    for t in range(1, 16):
        pltpu.sync_copy(shd.at[t, pl.ds(base, NSL)], tmp_v)

        def _acc(i, _):
            sl = pl.ds(pl.multiple_of(i * 16, 16), 16)
            dv_v[sl] = dv_v[sl] + tmp_v[sl]
            return 0

        lax.fori_loop(0, NSL // 16, _acc, 0)

    def _rs(i, _):
        sl = pl.ds(pl.multiple_of(i * 16, 16), 16)
        dv_v[sl] = _fast_rsqrt(dv_v[sl] + 1.0)
        return 0

    lax.fori_loop(0, NSL // 16, _rs, 0)

    @pl.when(c == 0)
    def _():
        pltpu.sync_copy(dv_v, dinv_hbm.at[pl.ds(base, NSL)])

    half = NSL // 2
    for h in range(2):
        r0 = base + h * half
        pltpu.sync_copy(
            x_hbm.at[pl.ds(r0, half), pl.ds(pl.multiple_of(c * 128, 128), 128)],
            xb_v)

        def _scale(i, _):
            ridx = jnp.full((16,), h * half + i, _I32)
            dsp = plsc.load_gather(dv_v, [ridx])
            for k in range(8):
                sl = pl.ds(k * 16, 16)
                xb_v[i, sl] = xb_v[i, sl] * dsp
            return 0

        lax.fori_loop(0, half, _scale, 0)

        @pl.when(c == 0)
        def _():
            pltpu.sync_copy(xb_v, xp0_hbm.at[pl.ds(r0, half)])

        @pl.when(c == 1)
        def _():
            pltpu.sync_copy(xb_v, xp1_hbm.at[pl.ds(r0, half)])


def _k2_body(xp0_hbm, xp1_hbm, src_hbm, dst_hbm, agg0_hbm, agg1_hbm,
             sidx_v, didx_v, buf0_v, buf1_v, acc_sh, sem0, sem1):
    c = lax.axis_index("c")
    s = lax.axis_index("s")
    bufs = (buf0_v, buf1_v)
    sems = (sem0, sem1)

    zeros16 = jnp.zeros((16,), _F32)

    def _zero(i, _):
        for k in range(8):
            buf0_v[i, pl.ds(k * 16, 16)] = zeros16
        return 0

    lax.fori_loop(0, 128, _zero, 0)

    for h in range(NSL // 128):
        pltpu.sync_copy(buf0_v, acc_sh.at[pl.ds(s * NSL + h * 128, 128)])
    plsc.subcore_barrier()

    def _start_gather(j, b):
        @pl.when(c == 0)
        def _():
            pltpu.async_copy(xp0_hbm.at[sidx_v.at[j]], bufs[b], sems[b])

        @pl.when(c == 1)
        def _():
            pltpu.async_copy(xp1_hbm.at[sidx_v.at[j]], bufs[b], sems[b])

    def _wait_gather(j, b):
        @pl.when(c == 0)
        def _():
            pltpu.make_async_copy(xp0_hbm.at[sidx_v.at[j]], bufs[b],
                                  sems[b]).wait()

        @pl.when(c == 1)
        def _():
            pltpu.make_async_copy(xp1_hbm.at[sidx_v.at[j]], bufs[b],
                                  sems[b]).wait()

    for ck in range(RPT // CHUNK):
        row0 = s * RPT + ck * CHUNK
        pltpu.sync_copy(src_hbm.at[pl.ds(row0, CHUNK)], sidx_v)
        pltpu.sync_copy(dst_hbm.at[pl.ds(row0, CHUNK)], didx_v)

        _start_gather(0, 0)
        _start_gather(1, 1)

        @pl.loop(0, CHUNK - 2, step=2)
        def _edge(g):
            for b in range(2):
                j = g + b
                _wait_gather(j, b)
                pltpu.sync_copy(bufs[b], acc_sh.at[didx_v.at[j]], add=True)
                _start_gather(j + 2, b)

        for b in range(2):
            j = CHUNK - 2 + b
            _wait_gather(j, b)
            pltpu.sync_copy(bufs[b], acc_sh.at[didx_v.at[j]], add=True)

    plsc.subcore_barrier()

    for h in range(NSL // 128):
        rows = pl.ds(s * NSL + h * 128, 128)
        pltpu.sync_copy(acc_sh.at[rows], buf0_v)

        @pl.when(c == 0)
        def _():
            pltpu.sync_copy(buf0_v, agg0_hbm.at[rows])

        @pl.when(c == 1)
        def _():
            pltpu.sync_copy(buf0_v, agg1_hbm.at[rows])


def _k3_body(agg0, agg1, xp0, xp1, dinv, w, b2, a2, u2, out_ref):
    w_ = w[...]
    u_ = u2[...]
    wv = jnp.dot(u_, w_, preferred_element_type=_F32)
    nv = jnp.sqrt(jnp.sum(wv * wv))
    v = wv / (nv + 1e-12)
    wv2 = lax.dot_general(v, w_, (((1,), (1,)), ((), ())),
                          preferred_element_type=_F32)
    nu = jnp.sqrt(jnp.sum(wv2 * wv2))
    sigma = jnp.sum(wv2 * wv2) / (nu + 1e-12)

    d = dinv[...]
    t0 = (agg0[...] + xp0[...]) * d
    t1 = (agg1[...] + xp1[...]) * d
    o = (jnp.dot(t0, w_[0:128, :], preferred_element_type=_F32)
         + jnp.dot(t1, w_[128:256, :], preferred_element_type=_F32))
    o = o * (1.0 / sigma) + b2[...]
    al = a2[0, 0]
    out_ref[...] = jnp.where(o >= 0, o, al * o)


@jax.jit
def kernel(x, edge_index, W, b, a, u):
    src = edge_index[0]
    dst = edge_index[1]
    pad = jnp.full((EP - E,), N, _I32)
    srcr = jnp.concatenate([src, pad]).reshape(ER, 128)
    dstr = jnp.concatenate([dst, pad]).reshape(ER, 128)
    x_pad = jnp.pad(x, ((0, NP - N), (0, 0)))

    mesh = plsc.VectorSubcoreMesh(core_axis_name="c", subcore_axis_name="s")

    k1 = pl.kernel(
        _k1_body,
        out_type=(
            jax.ShapeDtypeStruct((NP,), _F32),
            jax.ShapeDtypeStruct((NP, 128), _F32),
            jax.ShapeDtypeStruct((NP, 128), _F32),
        ),
        mesh=mesh,
        scratch_types=[
            pltpu.VMEM((RPT, 128), _I32),
            pltpu.VMEM((NP,), _F32),
            pltpu.VMEM((NSL,), _F32),
            pltpu.VMEM((NSL,), _F32),
            pltpu.VMEM((NSL // 2, 128), _F32),
            pltpu.VMEM_SHARED((16, NP), _F32),
        ],
        compiler_params=pltpu.CompilerParams(needs_layout_passes=False),
    )
    dinv, xp0, xp1 = k1(x_pad, dstr)

    k2 = pl.kernel(
        _k2_body,
        out_type=(
            jax.ShapeDtypeStruct((NP, 128), _F32),
            jax.ShapeDtypeStruct((NP, 128), _F32),
        ),
        mesh=mesh,
        scratch_types=[
            pltpu.VMEM((CHUNK, 128), _I32),
            pltpu.VMEM((CHUNK, 128), _I32),
            pltpu.VMEM((128, 128), _F32),
            pltpu.VMEM((128, 128), _F32),
            pltpu.VMEM_SHARED((NP, 128), _F32),
            pltpu.SemaphoreType.DMA,
            pltpu.SemaphoreType.DMA,
        ],
        compiler_params=pltpu.CompilerParams(needs_layout_passes=False),
    )
    agg0, agg1 = k2(xp0, xp1, srcr, dstr)

    blk = 512
    grid = NP // blk
    outp = pl.pallas_call(
        _k3_body,
        grid=(grid,),
        in_specs=[
            pl.BlockSpec((blk, 128), lambda i: (i, 0)),
            pl.BlockSpec((blk, 128), lambda i: (i, 0)),
            pl.BlockSpec((blk, 128), lambda i: (i, 0)),
            pl.BlockSpec((blk, 128), lambda i: (i, 0)),
            pl.BlockSpec((blk, 1), lambda i: (i, 0)),
            pl.BlockSpec((NF, NH), lambda i: (0, 0)),
            pl.BlockSpec((1, NH), lambda i: (0, 0)),
            pl.BlockSpec((1, 1), lambda i: (0, 0)),
            pl.BlockSpec((1, NF), lambda i: (0, 0)),
        ],
        out_specs=pl.BlockSpec((blk, NH), lambda i: (i, 0)),
        out_shape=jax.ShapeDtypeStruct((NP, NH), _F32),
    )(agg0, agg1, xp0, xp1, dinv.reshape(NP, 1), W,
      b.reshape(1, NH), a.reshape(1, 1), u.reshape(1, NF))

    return outp[:N]


# speedup vs baseline: 15.1871x; 1.2282x over previous
import functools

import jax
import jax.numpy as jnp
from jax import lax
from jax.experimental import pallas as pl
from jax.experimental.pallas import tpu as pltpu
from jax.experimental.pallas import tpu_sc as plsc

N = 10000
E = 160000
NF = 256
NH = 512

NP = 10240
EP = 163840
ER = EP // 128
RPT = ER // 16
NSL = NP // 16
EB = 64
ERW = EP // EB
SPT = ERW // 16
CHUNK = 32

_F32 = jnp.float32
_I32 = jnp.int32


def _fast_rsqrt(d):
    ih = plsc.bitcast(d, _I32)
    ih = jnp.int32(0x5F3759DF) - lax.shift_right_logical(ih, 1)
    y = plsc.bitcast(ih, _F32)
    for _ in range(3):
        y = y * (1.5 - 0.5 * d * y * y)
    return y


def _k1_body(x_hbm, dst_hbm, dinv_hbm, xp0_hbm, xp1_hbm,
             idx_v, deg_v, dv_v, tmp_v, xb_v, shd):
    c = lax.axis_index("c")
    s = lax.axis_index("s")
    base = s * NSL

    pltpu.sync_copy(dst_hbm.at[pl.ds(s * SPT, SPT)], idx_v)

    zeros16 = jnp.zeros((16,), _F32)

    def _zero(i, _):
        deg_v[pl.ds(pl.multiple_of(i * 16, 16), 16)] = zeros16
        return 0

    lax.fori_loop(0, NP // 16, _zero, 0)

    ones16 = jnp.ones((16,), _F32)

    def _scat(j, _):
        for k in range(EB // 16):
            iv = idx_v[j, pl.ds(k * 16, 16)]
            plsc.addupdate_scatter(deg_v, [iv], ones16)
        return 0

    lax.fori_loop(0, SPT, _scat, 0)

    pltpu.sync_copy(deg_v, shd.at[s])
    plsc.subcore_barrier()

    pltpu.sync_copy(shd.at[0, pl.ds(base, NSL)], dv_v)
    for t in range(1, 16):
        pltpu.sync_copy(shd.at[t, pl.ds(base, NSL)], tmp_v)

        def _acc(i, _):
            sl = pl.ds(pl.multiple_of(i * 16, 16), 16)
            dv_v[sl] = dv_v[sl] + tmp_v[sl]
            return 0

        lax.fori_loop(0, NSL // 16, _acc, 0)

    def _rs(i, _):
        sl = pl.ds(pl.multiple_of(i * 16, 16), 16)
        dv_v[sl] = _fast_rsqrt(dv_v[sl] + 1.0)
        return 0

    lax.fori_loop(0, NSL // 16, _rs, 0)

    @pl.when(c == 0)
    def _():
        pltpu.sync_copy(dv_v, dinv_hbm.at[pl.ds(base, NSL)])

    half = NSL // 2
    for h in range(2):
        r0 = base + h * half
        pltpu.sync_copy(
            x_hbm.at[pl.ds(r0, half), pl.ds(pl.multiple_of(c * 128, 128), 128)],
            xb_v)

        def _scale(i, _):
            ridx = jnp.full((16,), h * half + i, _I32)
            dsp = plsc.load_gather(dv_v, [ridx])
            for k in range(8):
                sl = pl.ds(k * 16, 16)
                xb_v[i, sl] = xb_v[i, sl] * dsp
            return 0

        lax.fori_loop(0, half, _scale, 0)

        @pl.when(c == 0)
        def _():
            pltpu.sync_copy(xb_v, xp0_hbm.at[pl.ds(r0, half)])

        @pl.when(c == 1)
        def _():
            pltpu.sync_copy(xb_v, xp1_hbm.at[pl.ds(r0, half)])


def _k2_body(xp0_hbm, xp1_hbm, src_hbm, dst_hbm, agg0_hbm, agg1_hbm,
             sidx_v, didx_v, buf0_v, buf1_v, buf2_v, buf3_v, acc_sh,
             gsem0, gsem1, gsem2, gsem3, ssem0, ssem1, ssem2, ssem3):
    c = lax.axis_index("c")
    s = lax.axis_index("s")
    bufs = (buf0_v, buf1_v, buf2_v, buf3_v)
    gsems = (gsem0, gsem1, gsem2, gsem3)
    ssems = (ssem0, ssem1, ssem2, ssem3)

    zeros16 = jnp.zeros((16,), _F32)

    def _zero(i, _):
        for k in range(8):
            buf0_v[i, pl.ds(k * 16, 16)] = zeros16
        return 0

    lax.fori_loop(0, EB, _zero, 0)

    for h in range(NSL // EB):
        pltpu.sync_copy(buf0_v, acc_sh.at[pl.ds(s * NSL + h * EB, EB)])
    plsc.subcore_barrier()

    def _start_gather(j, b):
        @pl.when(c == 0)
        def _():
            pltpu.async_copy(xp0_hbm.at[sidx_v.at[j]], bufs[b], gsems[b])

        @pl.when(c == 1)
        def _():
            pltpu.async_copy(xp1_hbm.at[sidx_v.at[j]], bufs[b], gsems[b])

    def _wait_gather(j, b):
        @pl.when(c == 0)
        def _():
            pltpu.make_async_copy(xp0_hbm.at[sidx_v.at[j]], bufs[b],
                                  gsems[b]).wait()

        @pl.when(c == 1)
        def _():
            pltpu.make_async_copy(xp1_hbm.at[sidx_v.at[j]], bufs[b],
                                  gsems[b]).wait()

    def _start_scatter(j, b):
        pltpu.async_copy(bufs[b], acc_sh.at[didx_v.at[j]], ssems[b],
                         add=True)

    def _wait_scatter(j, b):
        pltpu.make_async_copy(bufs[b], acc_sh.at[didx_v.at[j]],
                              ssems[b]).wait()

    for ck in range(SPT // CHUNK):
        row0 = s * SPT + ck * CHUNK
        pltpu.sync_copy(src_hbm.at[pl.ds(row0, CHUNK)], sidx_v)
        pltpu.sync_copy(dst_hbm.at[pl.ds(row0, CHUNK)], didx_v)

        _start_gather(0, 0)
        _start_gather(1, 1)
        _start_gather(2, 2)
        _wait_gather(0, 0)
        _start_scatter(0, 0)
        _start_gather(3, 3)

        @pl.loop(1, CHUNK - 3, step=4)
        def _edge(g):
            for db in range(4):
                j = g + db
                b = (1 + db) % 4
                _wait_gather(j, b)
                _start_scatter(j, b)
                _wait_scatter(j - 1, db % 4)
                _start_gather(j + 3, db % 4)

        for jj in range(CHUNK - 3, CHUNK):
            _wait_gather(jj, jj % 4)
            _start_scatter(jj, jj % 4)
            _wait_scatter(jj - 1, (jj - 1) % 4)
        _wait_scatter(CHUNK - 1, (CHUNK - 1) % 4)

    plsc.subcore_barrier()

    for h in range(NSL // EB):
        rows = pl.ds(s * NSL + h * EB, EB)
        pltpu.sync_copy(acc_sh.at[rows], buf0_v)

        @pl.when(c == 0)
        def _():
            pltpu.sync_copy(buf0_v, agg0_hbm.at[rows])

        @pl.when(c == 1)
        def _():
            pltpu.sync_copy(buf0_v, agg1_hbm.at[rows])


def _k3_body(agg0, agg1, xp0, xp1, dinv, w, b2, a2, u2, out_ref):
    w_ = w[...]
    u_ = u2[...]
    wv = jnp.dot(u_, w_, preferred_element_type=_F32)
    nv = jnp.sqrt(jnp.sum(wv * wv))
    v = wv / (nv + 1e-12)
    wv2 = lax.dot_general(v, w_, (((1,), (1,)), ((), ())),
                          preferred_element_type=_F32)
    nu = jnp.sqrt(jnp.sum(wv2 * wv2))
    sigma = jnp.sum(wv2 * wv2) / (nu + 1e-12)

    d = dinv[...]
    t0 = (agg0[...] + xp0[...]) * d
    t1 = (agg1[...] + xp1[...]) * d
    o = (jnp.dot(t0, w_[0:128, :], preferred_element_type=_F32)
         + jnp.dot(t1, w_[128:256, :], preferred_element_type=_F32))
    o = o * (1.0 / sigma) + b2[...]
    al = a2[0, 0]
    out_ref[...] = jnp.where(o >= 0, o, al * o)


@jax.jit
def kernel(x, edge_index, W, b, a, u):
    src = edge_index[0]
    dst = edge_index[1]
    pad = jnp.full((EP - E,), N, _I32)
    srcr = jnp.concatenate([src, pad]).reshape(ERW, EB)
    dstr = jnp.concatenate([dst, pad]).reshape(ERW, EB)
    x_pad = jnp.pad(x, ((0, NP - N), (0, 0)))

    mesh = plsc.VectorSubcoreMesh(core_axis_name="c", subcore_axis_name="s")

    k1 = pl.kernel(
        _k1_body,
        out_type=(
            jax.ShapeDtypeStruct((NP,), _F32),
            jax.ShapeDtypeStruct((NP, 128), _F32),
            jax.ShapeDtypeStruct((NP, 128), _F32),
        ),
        mesh=mesh,
        scratch_types=[
            pltpu.VMEM((SPT, EB), _I32),
            pltpu.VMEM((NP,), _F32),
            pltpu.VMEM((NSL,), _F32),
            pltpu.VMEM((NSL,), _F32),
            pltpu.VMEM((NSL // 2, 128), _F32),
            pltpu.VMEM_SHARED((16, NP), _F32),
        ],
        compiler_params=pltpu.CompilerParams(needs_layout_passes=False),
    )
    dinv, xp0, xp1 = k1(x_pad, dstr)

    k2 = pl.kernel(
        _k2_body,
        out_type=(
            jax.ShapeDtypeStruct((NP, 128), _F32),
            jax.ShapeDtypeStruct((NP, 128), _F32),
        ),
        mesh=mesh,
        scratch_types=[
            pltpu.VMEM((CHUNK, EB), _I32),
            pltpu.VMEM((CHUNK, EB), _I32),
            pltpu.VMEM((EB, 128), _F32),
            pltpu.VMEM((EB, 128), _F32),
            pltpu.VMEM((EB, 128), _F32),
            pltpu.VMEM((EB, 128), _F32),
            pltpu.VMEM_SHARED((NP, 128), _F32),
        ] + [pltpu.SemaphoreType.DMA] * 8,
        compiler_params=pltpu.CompilerParams(needs_layout_passes=False),
    )
    agg0, agg1 = k2(xp0, xp1, srcr, dstr)

    blk = 512
    grid = NP // blk
    outp = pl.pallas_call(
        _k3_body,
        grid=(grid,),
        in_specs=[
            pl.BlockSpec((blk, 128), lambda i: (i, 0)),
            pl.BlockSpec((blk, 128), lambda i: (i, 0)),
            pl.BlockSpec((blk, 128), lambda i: (i, 0)),
            pl.BlockSpec((blk, 128), lambda i: (i, 0)),
            pl.BlockSpec((blk, 1), lambda i: (i, 0)),
            pl.BlockSpec((NF, NH), lambda i: (0, 0)),
            pl.BlockSpec((1, NH), lambda i: (0, 0)),
            pl.BlockSpec((1, 1), lambda i: (0, 0)),
            pl.BlockSpec((1, NF), lambda i: (0, 0)),
        ],
        out_specs=pl.BlockSpec((blk, NH), lambda i: (i, 0)),
        out_shape=jax.ShapeDtypeStruct((NP, NH), _F32),
    )(agg0, agg1, xp0, xp1, dinv.reshape(NP, 1), W,
      b.reshape(1, NH), a.reshape(1, 1), u.reshape(1, NF))

    return outp[:N]


# speedup vs baseline: 15.6520x; 1.0306x over previous
import functools

import jax
import jax.numpy as jnp
from jax import lax
from jax.experimental import pallas as pl
from jax.experimental.pallas import tpu as pltpu
from jax.experimental.pallas import tpu_sc as plsc

N = 10000
E = 160000
NF = 256
NH = 512

NP = 10240
EP = 163840
ER = EP // 128
RPT = ER // 16
NSL = NP // 16
EB = 64
ERW = EP // EB
SPT = ERW // 16
CHUNK = 32

_F32 = jnp.float32
_I32 = jnp.int32


def _fast_rsqrt(d):
    ih = plsc.bitcast(d, _I32)
    ih = jnp.int32(0x5F3759DF) - lax.shift_right_logical(ih, 1)
    y = plsc.bitcast(ih, _F32)
    for _ in range(3):
        y = y * (1.5 - 0.5 * d * y * y)
    return y


def _k1_body(x_hbm, dst_hbm, dinv_hbm, xp0_hbm, xp1_hbm,
             idx_v, deg_v, dv_v, tmp_v, xb_v, shd):
    c = lax.axis_index("c")
    s = lax.axis_index("s")
    base = s * NSL

    pltpu.sync_copy(dst_hbm.at[pl.ds(s * SPT, SPT)], idx_v)

    zeros16 = jnp.zeros((16,), _F32)

    def _zero(i, _):
        deg_v[pl.ds(pl.multiple_of(i * 16, 16), 16)] = zeros16
        return 0

    lax.fori_loop(0, NP // 16, _zero, 0)

    ones16 = jnp.ones((16,), _F32)

    def _scat(j, _):
        for k in range(EB // 16):
            iv = idx_v[j, pl.ds(k * 16, 16)]
            plsc.addupdate_scatter(deg_v, [iv], ones16)
        return 0

    lax.fori_loop(0, SPT, _scat, 0)

    pltpu.sync_copy(deg_v, shd.at[s])
    plsc.subcore_barrier()

    pltpu.sync_copy(shd.at[pl.ds(0, 16), pl.ds(base, NSL)], tmp_v)

    def _acc(i, _):
        sl = pl.ds(pl.multiple_of(i * 16, 16), 16)
        acc = tmp_v[0, sl]
        for t in range(1, 16):
            acc = acc + tmp_v[t, sl]
        dv_v[sl] = acc
        return 0

    lax.fori_loop(0, NSL // 16, _acc, 0)

    def _rs(i, _):
        sl = pl.ds(pl.multiple_of(i * 16, 16), 16)
        dv_v[sl] = _fast_rsqrt(dv_v[sl] + 1.0)
        return 0

    lax.fori_loop(0, NSL // 16, _rs, 0)

    @pl.when(c == 0)
    def _():
        pltpu.sync_copy(dv_v, dinv_hbm.at[pl.ds(base, NSL)])

    half = NSL // 2
    for h in range(2):
        r0 = base + h * half
        pltpu.sync_copy(
            x_hbm.at[pl.ds(r0, half), pl.ds(pl.multiple_of(c * 128, 128), 128)],
            xb_v)

        def _scale(i, _):
            ridx = jnp.full((16,), h * half + i, _I32)
            dsp = plsc.load_gather(dv_v, [ridx])
            for k in range(8):
                sl = pl.ds(k * 16, 16)
                xb_v[i, sl] = xb_v[i, sl] * dsp
            return 0

        lax.fori_loop(0, half, _scale, 0)

        @pl.when(c == 0)
        def _():
            pltpu.sync_copy(xb_v, xp0_hbm.at[pl.ds(r0, half)])

        @pl.when(c == 1)
        def _():
            pltpu.sync_copy(xb_v, xp1_hbm.at[pl.ds(r0, half)])


def _k2_body(xp0_hbm, xp1_hbm, src_hbm, dst_hbm, agg0_hbm, agg1_hbm,
             sidx0_v, didx0_v, sidx1_v, didx1_v,
             buf0_v, buf1_v, buf2_v, buf3_v, acc_sh,
             gsem0, gsem1, gsem2, gsem3, ssem0, ssem1, ssem2, ssem3,
             isem0, isem1):
    c = lax.axis_index("c")
    s = lax.axis_index("s")
    bufs = (buf0_v, buf1_v, buf2_v, buf3_v)
    gsems = (gsem0, gsem1, gsem2, gsem3)
    ssems = (ssem0, ssem1, ssem2, ssem3)
    sidxs = (sidx0_v, sidx1_v)
    didxs = (didx0_v, didx1_v)
    isems = (isem0, isem1)

    zeros16 = jnp.zeros((16,), _F32)

    def _zero(i, _):
        for k in range(8):
            buf0_v[i, pl.ds(k * 16, 16)] = zeros16
        return 0

    lax.fori_loop(0, EB, _zero, 0)

    for h in range(NSL // EB):
        pltpu.sync_copy(buf0_v, acc_sh.at[pl.ds(s * NSL + h * EB, EB)])
    plsc.subcore_barrier()

    def _start_gather(j, b, iv):
        @pl.when(c == 0)
        def _():
            pltpu.async_copy(xp0_hbm.at[iv.at[j]], bufs[b], gsems[b])

        @pl.when(c == 1)
        def _():
            pltpu.async_copy(xp1_hbm.at[iv.at[j]], bufs[b], gsems[b])

    def _wait_gather(j, b, iv):
        @pl.when(c == 0)
        def _():
            pltpu.make_async_copy(xp0_hbm.at[iv.at[j]], bufs[b],
                                  gsems[b]).wait()

        @pl.when(c == 1)
        def _():
            pltpu.make_async_copy(xp1_hbm.at[iv.at[j]], bufs[b],
                                  gsems[b]).wait()

    def _start_scatter(j, b, iv):
        pltpu.async_copy(bufs[b], acc_sh.at[iv.at[j]], ssems[b], add=True)

    def _wait_scatter(j, b, iv):
        pltpu.make_async_copy(bufs[b], acc_sh.at[iv.at[j]], ssems[b]).wait()

    def _stage_idx(ck, sync):
        row0 = s * SPT + ck * CHUNK
        p = ck % 2
        if sync:
            pltpu.sync_copy(src_hbm.at[pl.ds(row0, CHUNK)], sidxs[p])
            pltpu.sync_copy(dst_hbm.at[pl.ds(row0, CHUNK)], didxs[p])
        else:
            pltpu.async_copy(src_hbm.at[pl.ds(row0, CHUNK)], sidxs[p],
                             isems[0])
            pltpu.async_copy(dst_hbm.at[pl.ds(row0, CHUNK)], didxs[p],
                             isems[1])

    def _wait_idx(ck):
        row0 = s * SPT + ck * CHUNK
        p = ck % 2
        pltpu.make_async_copy(src_hbm.at[pl.ds(row0, CHUNK)], sidxs[p],
                              isems[0]).wait()
        pltpu.make_async_copy(dst_hbm.at[pl.ds(row0, CHUNK)], didxs[p],
                              isems[1]).wait()

    NCK = SPT // CHUNK
    _stage_idx(0, True)
    _start_gather(0, 0, sidxs[0])
    _start_gather(1, 1, sidxs[0])
    _start_gather(2, 2, sidxs[0])

    for ck in range(NCK):
        si = sidxs[ck % 2]
        di = didxs[ck % 2]
        _wait_gather(0, 0, si)
        _start_scatter(0, 0, di)
        if ck > 0:
            _wait_scatter(CHUNK - 1, 3, didxs[1 - ck % 2])
        if ck < NCK - 1:
            _stage_idx(ck + 1, False)
        _start_gather(3, 3, si)

        @pl.loop(1, CHUNK - 3, step=4)
        def _edge(g):
            for db in range(4):
                j = g + db
                b = (1 + db) % 4
                _wait_gather(j, b, si)
                _start_scatter(j, b, di)
                _wait_scatter(j - 1, db % 4, di)
                _start_gather(j + 3, db % 4, si)

        for jj in range(CHUNK - 3, CHUNK):
            _wait_gather(jj, jj % 4, si)
            _start_scatter(jj, jj % 4, di)
            _wait_scatter(jj - 1, (jj - 1) % 4, di)
        if ck < NCK - 1:
            _wait_idx(ck + 1)
            nsi = sidxs[(ck + 1) % 2]
            _start_gather(0, 0, nsi)
            _start_gather(1, 1, nsi)
            _start_gather(2, 2, nsi)
    _wait_scatter(CHUNK - 1, 3, didxs[(NCK - 1) % 2])

    plsc.subcore_barrier()

    for h in range(NSL // EB):
        rows = pl.ds(s * NSL + h * EB, EB)
        pltpu.sync_copy(acc_sh.at[rows], buf0_v)

        @pl.when(c == 0)
        def _():
            pltpu.sync_copy(buf0_v, agg0_hbm.at[rows])

        @pl.when(c == 1)
        def _():
            pltpu.sync_copy(buf0_v, agg1_hbm.at[rows])


def _k3_body(agg0, agg1, xp0, xp1, dinv, w, b2, a2, u2, out_ref):
    w_ = w[...]
    u_ = u2[...]
    wv = jnp.dot(u_, w_, preferred_element_type=_F32)
    nv = jnp.sqrt(jnp.sum(wv * wv))
    v = wv / (nv + 1e-12)
    wv2 = lax.dot_general(v, w_, (((1,), (1,)), ((), ())),
                          preferred_element_type=_F32)
    nu = jnp.sqrt(jnp.sum(wv2 * wv2))
    sigma = jnp.sum(wv2 * wv2) / (nu + 1e-12)

    d = dinv[...]
    t0 = (agg0[...] + xp0[...]) * d
    t1 = (agg1[...] + xp1[...]) * d
    o = (jnp.dot(t0, w_[0:128, :], preferred_element_type=_F32)
         + jnp.dot(t1, w_[128:256, :], preferred_element_type=_F32))
    o = o * (1.0 / sigma) + b2[...]
    al = a2[0, 0]
    out_ref[...] = jnp.where(o >= 0, o, al * o)


@jax.jit
def kernel(x, edge_index, W, b, a, u):
    src = edge_index[0]
    dst = edge_index[1]
    pad = jnp.full((EP - E,), N, _I32)
    srcr = jnp.concatenate([src, pad]).reshape(ERW, EB)
    dstr = jnp.concatenate([dst, pad]).reshape(ERW, EB)
    x_pad = jnp.pad(x, ((0, NP - N), (0, 0)))

    mesh = plsc.VectorSubcoreMesh(core_axis_name="c", subcore_axis_name="s")

    k1 = pl.kernel(
        _k1_body,
        out_type=(
            jax.ShapeDtypeStruct((NP,), _F32),
            jax.ShapeDtypeStruct((NP, 128), _F32),
            jax.ShapeDtypeStruct((NP, 128), _F32),
        ),
        mesh=mesh,
        scratch_types=[
            pltpu.VMEM((SPT, EB), _I32),
            pltpu.VMEM((NP,), _F32),
            pltpu.VMEM((NSL,), _F32),
            pltpu.VMEM((16, NSL), _F32),
            pltpu.VMEM((NSL // 2, 128), _F32),
            pltpu.VMEM_SHARED((16, NP), _F32),
        ],
        compiler_params=pltpu.CompilerParams(needs_layout_passes=False),
    )
    dinv, xp0, xp1 = k1(x_pad, dstr)

    k2 = pl.kernel(
        _k2_body,
        out_type=(
            jax.ShapeDtypeStruct((NP, 128), _F32),
            jax.ShapeDtypeStruct((NP, 128), _F32),
        ),
        mesh=mesh,
        scratch_types=[
            pltpu.VMEM((CHUNK, EB), _I32),
            pltpu.VMEM((CHUNK, EB), _I32),
            pltpu.VMEM((CHUNK, EB), _I32),
            pltpu.VMEM((CHUNK, EB), _I32),
            pltpu.VMEM((EB, 128), _F32),
            pltpu.VMEM((EB, 128), _F32),
            pltpu.VMEM((EB, 128), _F32),
            pltpu.VMEM((EB, 128), _F32),
            pltpu.VMEM_SHARED((NP, 128), _F32),
        ] + [pltpu.SemaphoreType.DMA] * 10,
        compiler_params=pltpu.CompilerParams(needs_layout_passes=False),
    )
    agg0, agg1 = k2(xp0, xp1, srcr, dstr)

    blk = 512
    grid = NP // blk
    outp = pl.pallas_call(
        _k3_body,
        grid=(grid,),
        in_specs=[
            pl.BlockSpec((blk, 128), lambda i: (i, 0)),
            pl.BlockSpec((blk, 128), lambda i: (i, 0)),
            pl.BlockSpec((blk, 128), lambda i: (i, 0)),
            pl.BlockSpec((blk, 128), lambda i: (i, 0)),
            pl.BlockSpec((blk, 1), lambda i: (i, 0)),
            pl.BlockSpec((NF, NH), lambda i: (0, 0)),
            pl.BlockSpec((1, NH), lambda i: (0, 0)),
            pl.BlockSpec((1, 1), lambda i: (0, 0)),
            pl.BlockSpec((1, NF), lambda i: (0, 0)),
        ],
        out_specs=pl.BlockSpec((blk, NH), lambda i: (i, 0)),
        out_shape=jax.ShapeDtypeStruct((NP, NH), _F32),
    )(agg0, agg1, xp0, xp1, dinv.reshape(NP, 1), W,
      b.reshape(1, NH), a.reshape(1, 1), u.reshape(1, NF))

    return outp[:N]
